# Initial kernel scaffold; baseline (speedup 1.0000x reference)
import jax
import jax.numpy as jnp
from jax.experimental import pallas as pl


def kernel(x, W_enc, b_enc, W_dec, b_dec):
    raise NotImplementedError("write your pallas kernel here")


# speedup vs baseline: 9.4280x; 9.4280x over previous
import jax
import jax.numpy as jnp
from jax.experimental import pallas as pl

_TOPK = 64


def _enc_kernel(x_ref, we_ref, be_ref, bd_ref, pre_ref):
    xb = x_ref[...] - bd_ref[...]
    acc = jnp.dot(xb, we_ref[...], preferred_element_type=jnp.float32)
    pre_ref[...] = jnp.maximum(acc + be_ref[...], 0.0)


def _thr_kernel(pre_ref, thr_ref):
    bits = jax.lax.bitcast_convert_type(pre_ref[...], jnp.int32)
    bm = bits.shape[0]
    lo = jnp.zeros((bm, 1), jnp.int32)
    hi = jnp.full((bm, 1), 0x7F800000, jnp.int32)

    def body(_, carry):
        lo, hi = carry
        mid = lo + (hi - lo) // 2
        cnt = jnp.sum((bits >= mid).astype(jnp.float32), axis=1,
                      keepdims=True)
        ge = cnt >= _TOPK
        return jnp.where(ge, mid, lo), jnp.where(ge, hi, mid)

    lo, hi = jax.lax.fori_loop(0, 31, body, (lo, hi))
    thr_ref[...] = jax.lax.bitcast_convert_type(lo, jnp.float32)


def _dec_kernel(pre_ref, thr_ref, wd_ref, bd_ref, out_ref):
    j = pl.program_id(1)
    p = pre_ref[...]
    z = jnp.where(p >= thr_ref[...], p, 0.0)
    acc = jnp.dot(z, wd_ref[...], preferred_element_type=jnp.float32)

    @pl.when(j == 0)
    def _init():
        out_ref[...] = acc + bd_ref[...]

    @pl.when(j != 0)
    def _acc():
        out_ref[...] += acc


def kernel(x, W_enc, b_enc, W_dec, b_dec):
    b, s, d_model = x.shape
    m = b * s
    d_sae = W_enc.shape[1]
    x2 = x.reshape(m, d_model)
    be2 = b_enc.reshape(1, d_sae)
    bd2 = b_dec.reshape(1, d_model)

    bm1 = min(512, m)
    bn1 = min(512, d_sae)
    pre = pl.pallas_call(
        _enc_kernel,
        grid=(m // bm1, d_sae // bn1),
        in_specs=[
            pl.BlockSpec((bm1, d_model), lambda i, j: (i, 0)),
            pl.BlockSpec((d_model, bn1), lambda i, j: (0, j)),
            pl.BlockSpec((1, bn1), lambda i, j: (0, j)),
            pl.BlockSpec((1, d_model), lambda i, j: (0, 0)),
        ],
        out_specs=pl.BlockSpec((bm1, bn1), lambda i, j: (i, j)),
        out_shape=jax.ShapeDtypeStruct((m, d_sae), jnp.float32),
    )(x2, W_enc, be2, bd2)

    bm2 = min(128, m)
    thr = pl.pallas_call(
        _thr_kernel,
        grid=(m // bm2,),
        in_specs=[pl.BlockSpec((bm2, d_sae), lambda i: (i, 0))],
        out_specs=pl.BlockSpec((bm2, 1), lambda i: (i, 0)),
        out_shape=jax.ShapeDtypeStruct((m, 1), jnp.float32),
    )(pre)

    bm3 = min(512, m)
    bk3 = min(512, d_sae)
    x_hat = pl.pallas_call(
        _dec_kernel,
        grid=(m // bm3, d_sae // bk3),
        in_specs=[
            pl.BlockSpec((bm3, bk3), lambda i, j: (i, j)),
            pl.BlockSpec((bm3, 1), lambda i, j: (i, 0)),
            pl.BlockSpec((bk3, d_model), lambda i, j: (j, 0)),
            pl.BlockSpec((1, d_model), lambda i, j: (0, 0)),
        ],
        out_specs=pl.BlockSpec((bm3, d_model), lambda i, j: (i, 0)),
        out_shape=jax.ShapeDtypeStruct((m, d_model), jnp.float32),
    )(pre, thr, W_dec, bd2)

    return x_hat.reshape(b, s, d_model)
